# baseline (speedup 1.0000x reference)
import functools

import jax
import jax.numpy as jnp
from jax.experimental import pallas as pl
from jax.experimental.pallas import tpu as pltpu

PATCH = 7
BLK = 1024


def _round_up(v, m):
    return ((v + m - 1) // m) * m


def _stage_rows(src_ref, dst, d, w, npad):
    for dd in range(d):
        ch, rem = divmod(dd, PATCH * PATCH)
        dy, dx = divmod(rem, PATCH)
        off = dy * w + dx
        dst[pl.ds(dd, 1), :] = src_ref[0, pl.ds(ch, 1), pl.ds(off, npad)]


def _norms(stag, d, npad):
    outs = []
    for kb in range(npad // BLK):
        chunk = stag[:, pl.ds(kb * BLK, BLK)]
        outs.append(jnp.sum(chunk * chunk, axis=0, keepdims=True))
    return jnp.concatenate(outs, axis=1)


def _body(xf_ref, yf_ref, out_ref, sxd, syd, sxn, sy, mins, sprev, acc,
          *, c, w, d, npad, npos, ni, nt, nb, scale):
    b = pl.program_id(0)
    i = pl.program_id(1)
    t = pl.program_id(2)

    lane = jax.lax.broadcasted_iota(jnp.int32, (1, npad), 1)
    lane_valid = (jnp.remainder(lane, w) < (w - PATCH + 1)) & (lane < npos)

    @pl.when((b == 0) & (i == 0) & (t == 0))
    def _init_acc():
        acc[0, 0] = jnp.float32(0.0)

    @pl.when((i == 0) & (t == 0))
    def _build():
        _stage_rows(yf_ref, syd, d, w, npad)
        _stage_rows(xf_ref, sxd, d, w, npad)
        for kb in range(npad // BLK):
            sy[pl.ds(0, d), pl.ds(kb * BLK, BLK)] = (
                syd[:, pl.ds(kb * BLK, BLK)] * -2.0).astype(jnp.bfloat16)
        nyf = jnp.where(lane_valid, _norms(syd, d, npad), jnp.float32(1e30))
        ny_hi = nyf.astype(jnp.bfloat16)
        ny_lo = (nyf - ny_hi.astype(jnp.float32)).astype(jnp.bfloat16)
        sy[pl.ds(d, 1), :] = ny_hi
        sy[pl.ds(d + 1, 1), :] = ny_lo
        nx = _norms(sxd, d, npad)
        acc[0, 0] += jnp.sum(jnp.where(lane_valid, nx, 0.0))
        for ib in range(ni):
            blk = sxd[:, pl.ds(ib * BLK, BLK)]
            sxn[pl.ds(ib * BLK, BLK), pl.ds(0, d)] = (
                jnp.transpose(blk, (1, 0)).astype(jnp.bfloat16))
        sxn[:, pl.ds(d, 2)] = jnp.ones((npad, 2), jnp.bfloat16)

    @pl.when(t == 0)
    def _init_min():
        mins[...] = jnp.full_like(mins, jnp.inf)

    def _tree_min(blk):
        m = None
        for k in range(BLK // 128):
            part = blk[:, k * 128:(k + 1) * 128]
            m = part if m is None else jnp.minimum(m, part)
        return m

    xb = sxn[pl.ds(pl.multiple_of(i * BLK, BLK), BLK), :]
    yb = sy[:, pl.ds(pl.multiple_of(t * BLK, BLK), BLK)]
    s = jax.lax.dot_general(
        xb, yb, (((1,), (0,)), ((), ())),
        preferred_element_type=jnp.float32)

    @pl.when(t > 0)
    def _min_prev():
        mins[...] = jnp.minimum(mins[...], _tree_min(sprev[...]))

    sprev[...] = s

    @pl.when(t == nt - 1)
    def _accumulate():
        mcur = jnp.minimum(mins[...], _tree_min(s))
        mrow = jnp.min(mcur, axis=1, keepdims=True)
        pos = jax.lax.broadcasted_iota(jnp.int32, (BLK, 1), 0) + i * BLK
        valid = (jnp.remainder(pos, w) < (w - PATCH + 1)) & (pos < npos)
        acc[0, 0] += jnp.sum(jnp.where(valid, mrow, 0.0))

    @pl.when((b == nb - 1) & (i == ni - 1) & (t == nt - 1))
    def _finalize():
        out_ref[...] = jnp.full((1, 1), acc[0, 0] * scale, jnp.float32)


@jax.jit
def kernel(x, y):
    b, c, h, w = x.shape
    p = PATCH
    oh, ow = h - p + 1, w - p + 1
    d = c * p * p
    n_real = oh * ow
    npos = oh * w
    npad = _round_up(npos, BLK)
    flatpad = _round_up(npad + (p - 1) * w + p, 128)

    xf = jnp.pad(x.reshape(b, c, h * w), ((0, 0), (0, 0), (0, flatpad - h * w)))
    yf = jnp.pad(y.reshape(b, c, h * w), ((0, 0), (0, 0), (0, flatpad - h * w)))

    ni = npad // BLK
    nt = npad // BLK
    scale = 1.0 / (b * n_real * d)

    body = functools.partial(
        _body, c=c, w=w, d=d, npad=npad, npos=npos,
        ni=ni, nt=nt, nb=b, scale=scale)
    out = pl.pallas_call(
        body,
        grid=(b, ni, nt),
        in_specs=[
            pl.BlockSpec((1, c, flatpad), lambda bi, ii, ti: (bi, 0, 0)),
            pl.BlockSpec((1, c, flatpad), lambda bi, ii, ti: (bi, 0, 0)),
        ],
        out_specs=pl.BlockSpec((1, 1), lambda bi, ii, ti: (0, 0)),
        out_shape=jax.ShapeDtypeStruct((1, 1), jnp.float32),
        scratch_shapes=[
            pltpu.VMEM((d, npad), jnp.float32),
            pltpu.VMEM((d, npad), jnp.float32),
            pltpu.VMEM((npad, d + 2), jnp.bfloat16),
            pltpu.VMEM((d + 2, npad), jnp.bfloat16),
            pltpu.VMEM((BLK, 128), jnp.float32),
            pltpu.VMEM((BLK, BLK), jnp.float32),
            pltpu.SMEM((1, 1), jnp.float32),
        ],
        compiler_params=pltpu.CompilerParams(
            dimension_semantics=("arbitrary", "arbitrary", "arbitrary"),
        ),
    )(xf, yf)
    return out[0, 0]


# speedup vs baseline: 1.4113x; 1.4113x over previous
import functools

import jax
import jax.numpy as jnp
from jax.experimental import pallas as pl
from jax.experimental.pallas import tpu as pltpu

PATCH = 7
BLK = 1024


def _round_up(v, m):
    return ((v + m - 1) // m) * m


def _stage_rows(src_ref, dst, d, w, npad):
    for dd in range(d):
        ch, rem = divmod(dd, PATCH * PATCH)
        dy, dx = divmod(rem, PATCH)
        off = dy * w + dx
        dst[pl.ds(dd, 1), :] = src_ref[0, pl.ds(ch, 1), pl.ds(off, npad)]


def _norms(stag, d, npad):
    outs = []
    for kb in range(npad // BLK):
        chunk = stag[:, pl.ds(kb * BLK, BLK)]
        outs.append(jnp.sum(chunk * chunk, axis=0, keepdims=True))
    return jnp.concatenate(outs, axis=1)


def _body(xf_ref, yf_ref, out_ref, sxd, syd, sxn, sy, mins, acc,
          *, c, w, d, npad, npos, ni, nt, nb, scale):
    b = pl.program_id(0)
    i = pl.program_id(1)
    t = pl.program_id(2)

    lane = jax.lax.broadcasted_iota(jnp.int32, (1, npad), 1)
    lane_valid = (jnp.remainder(lane, w) < (w - PATCH + 1)) & (lane < npos)

    @pl.when((b == 0) & (i == 0) & (t == 0))
    def _init_acc():
        acc[0, 0] = jnp.float32(0.0)

    @pl.when((i == 0) & (t == 0))
    def _build():
        _stage_rows(yf_ref, syd, d, w, npad)
        _stage_rows(xf_ref, sxd, d, w, npad)
        for kb in range(npad // BLK):
            sy[pl.ds(0, d), pl.ds(kb * BLK, BLK)] = (
                syd[:, pl.ds(kb * BLK, BLK)] * -2.0).astype(jnp.bfloat16)
        nyf = jnp.where(lane_valid, _norms(syd, d, npad), jnp.float32(1e30))
        ny_hi = nyf.astype(jnp.bfloat16)
        ny_lo = (nyf - ny_hi.astype(jnp.float32)).astype(jnp.bfloat16)
        sy[pl.ds(d, 1), :] = ny_hi
        sy[pl.ds(d + 1, 1), :] = ny_lo
        nx = _norms(sxd, d, npad)
        acc[0, 0] += jnp.sum(jnp.where(lane_valid, nx, 0.0))
        for ib in range(ni):
            blk = sxd[:, pl.ds(ib * BLK, BLK)]
            sxn[pl.ds(ib * BLK, BLK), pl.ds(0, d)] = (
                jnp.transpose(blk, (1, 0)).astype(jnp.bfloat16))
        sxn[:, pl.ds(d, 2)] = jnp.ones((npad, 2), jnp.bfloat16)

    @pl.when(t == 0)
    def _init_min():
        mins[...] = jnp.full_like(mins, jnp.inf)

    def _tree_min(blk):
        m = None
        for k in range(blk.shape[1] // 128):
            part = blk[:, k * 128:(k + 1) * 128]
            m = part if m is None else jnp.minimum(m, part)
        return m

    xb = sxn[pl.ds(pl.multiple_of(i * BLK, BLK), BLK), :]
    half = BLK // 2
    yb0 = sy[:, pl.ds(pl.multiple_of(t * BLK, BLK), half)]
    yb1 = sy[:, pl.ds(pl.multiple_of(t * BLK + half, half), half)]
    s0 = jax.lax.dot_general(
        xb, yb0, (((1,), (0,)), ((), ())),
        preferred_element_type=jnp.float32)
    s1 = jax.lax.dot_general(
        xb, yb1, (((1,), (0,)), ((), ())),
        preferred_element_type=jnp.float32)
    m = jnp.minimum(_tree_min(s0), _tree_min(s1))
    mins[...] = jnp.minimum(mins[...], m)

    @pl.when(t == nt - 1)
    def _accumulate():
        mrow = jnp.min(mins[...], axis=1, keepdims=True)
        pos = jax.lax.broadcasted_iota(jnp.int32, (BLK, 1), 0) + i * BLK
        valid = (jnp.remainder(pos, w) < (w - PATCH + 1)) & (pos < npos)
        acc[0, 0] += jnp.sum(jnp.where(valid, mrow, 0.0))

    @pl.when((b == nb - 1) & (i == ni - 1) & (t == nt - 1))
    def _finalize():
        out_ref[...] = jnp.full((1, 1), acc[0, 0] * scale, jnp.float32)


@jax.jit
def kernel(x, y):
    b, c, h, w = x.shape
    p = PATCH
    oh, ow = h - p + 1, w - p + 1
    d = c * p * p
    n_real = oh * ow
    npos = oh * w
    npad = _round_up(npos, BLK)
    flatpad = _round_up(npad + (p - 1) * w + p, 128)

    xf = jnp.pad(x.reshape(b, c, h * w), ((0, 0), (0, 0), (0, flatpad - h * w)))
    yf = jnp.pad(y.reshape(b, c, h * w), ((0, 0), (0, 0), (0, flatpad - h * w)))

    ni = npad // BLK
    nt = npad // BLK
    scale = 1.0 / (b * n_real * d)

    body = functools.partial(
        _body, c=c, w=w, d=d, npad=npad, npos=npos,
        ni=ni, nt=nt, nb=b, scale=scale)
    out = pl.pallas_call(
        body,
        grid=(b, ni, nt),
        in_specs=[
            pl.BlockSpec((1, c, flatpad), lambda bi, ii, ti: (bi, 0, 0)),
            pl.BlockSpec((1, c, flatpad), lambda bi, ii, ti: (bi, 0, 0)),
        ],
        out_specs=pl.BlockSpec((1, 1), lambda bi, ii, ti: (0, 0)),
        out_shape=jax.ShapeDtypeStruct((1, 1), jnp.float32),
        scratch_shapes=[
            pltpu.VMEM((d, npad), jnp.float32),
            pltpu.VMEM((d, npad), jnp.float32),
            pltpu.VMEM((npad, d + 2), jnp.bfloat16),
            pltpu.VMEM((d + 2, npad), jnp.bfloat16),
            pltpu.VMEM((BLK, 128), jnp.float32),
            pltpu.SMEM((1, 1), jnp.float32),
        ],
        compiler_params=pltpu.CompilerParams(
            dimension_semantics=("arbitrary", "arbitrary", "arbitrary"),
        ),
    )(xf, yf)
    return out[0, 0]


# speedup vs baseline: 1.9527x; 1.3836x over previous
import functools

import jax
import jax.numpy as jnp
from jax.experimental import pallas as pl
from jax.experimental.pallas import tpu as pltpu

PATCH = 7
BLK = 1024


def _round_up(v, m):
    return ((v + m - 1) // m) * m


def _stage_rows(src_ref, dst, d, w, npad):
    for dd in range(d):
        ch, rem = divmod(dd, PATCH * PATCH)
        dy, dx = divmod(rem, PATCH)
        off = dy * w + dx
        dst[pl.ds(dd, 1), :] = src_ref[0, pl.ds(ch, 1), pl.ds(off, npad)]


def _norms(stag, d, npad):
    outs = []
    for kb in range(npad // BLK):
        chunk = stag[:, pl.ds(kb * BLK, BLK)]
        outs.append(jnp.sum(chunk * chunk, axis=0, keepdims=True))
    return jnp.concatenate(outs, axis=1)


def _body(xf_ref, yf_ref, out_ref, sxd, syd, sxn, sy, mins, acc,
          *, c, w, d, npad, npos, ni, nt, nb, scale):
    b = pl.program_id(0)
    i = pl.program_id(1)
    t = pl.program_id(2)

    lane = jax.lax.broadcasted_iota(jnp.int32, (1, npad), 1)
    lane_valid = (jnp.remainder(lane, w) < (w - PATCH + 1)) & (lane < npos)

    @pl.when((b == 0) & (i == 0) & (t == 0))
    def _init_acc():
        acc[0, 0] = jnp.float32(0.0)

    @pl.when((i == 0) & (t == 0))
    def _build():
        _stage_rows(yf_ref, syd, d, w, npad)
        _stage_rows(xf_ref, sxd, d, w, npad)
        for kb in range(npad // BLK):
            sy[pl.ds(0, d), pl.ds(kb * BLK, BLK)] = (
                syd[:, pl.ds(kb * BLK, BLK)] * -2.0).astype(jnp.bfloat16)
        nyf = jnp.where(lane_valid, _norms(syd, d, npad), jnp.float32(1e30))
        ny_hi = nyf.astype(jnp.bfloat16)
        ny_lo = (nyf - ny_hi.astype(jnp.float32)).astype(jnp.bfloat16)
        sy[pl.ds(d, 1), :] = ny_hi
        sy[pl.ds(d + 1, 1), :] = ny_lo
        nx = _norms(sxd, d, npad)
        acc[0, 0] += jnp.sum(jnp.where(lane_valid, nx, 0.0))
        for ib in range(ni):
            blk = sxd[:, pl.ds(ib * BLK, BLK)]
            sxn[pl.ds(ib * BLK, BLK), pl.ds(0, d)] = (
                jnp.transpose(blk, (1, 0)).astype(jnp.bfloat16))
        sxn[:, pl.ds(d, 2)] = jnp.ones((npad, 2), jnp.bfloat16)

    @pl.when(t == 0)
    def _init_min():
        mins[...] = jnp.full_like(mins, jnp.inf)

    def _tree_min(blk):
        m = None
        for k in range(blk.shape[1] // 128):
            part = blk[:, k * 128:(k + 1) * 128]
            m = part if m is None else jnp.minimum(m, part)
        return m

    bt = npad // nt
    xb = sxn[pl.ds(pl.multiple_of(i * BLK, BLK), BLK), :]
    yb = sy[:, pl.ds(pl.multiple_of(t * bt, bt), bt)]
    s = jax.lax.dot_general(
        xb, yb, (((1,), (0,)), ((), ())),
        preferred_element_type=jnp.float32)
    mins[...] = jnp.minimum(mins[...], _tree_min(s))

    @pl.when(t == nt - 1)
    def _accumulate():
        mrow = jnp.min(mins[...], axis=1, keepdims=True)
        pos = jax.lax.broadcasted_iota(jnp.int32, (BLK, 1), 0) + i * BLK
        valid = (jnp.remainder(pos, w) < (w - PATCH + 1)) & (pos < npos)
        acc[0, 0] += jnp.sum(jnp.where(valid, mrow, 0.0))

    @pl.when((b == nb - 1) & (i == ni - 1) & (t == nt - 1))
    def _finalize():
        out_ref[...] = jnp.full((1, 1), acc[0, 0] * scale, jnp.float32)


@jax.jit
def kernel(x, y):
    b, c, h, w = x.shape
    p = PATCH
    oh, ow = h - p + 1, w - p + 1
    d = c * p * p
    n_real = oh * ow
    npos = oh * w
    npad = _round_up(npos, BLK)
    flatpad = _round_up(npad + (p - 1) * w + p, 128)

    xf = jnp.pad(x.reshape(b, c, h * w), ((0, 0), (0, 0), (0, flatpad - h * w)))
    yf = jnp.pad(y.reshape(b, c, h * w), ((0, 0), (0, 0), (0, flatpad - h * w)))

    ni = npad // BLK
    nt = 2
    scale = 1.0 / (b * n_real * d)

    body = functools.partial(
        _body, c=c, w=w, d=d, npad=npad, npos=npos,
        ni=ni, nt=nt, nb=b, scale=scale)
    out = pl.pallas_call(
        body,
        grid=(b, ni, nt),
        in_specs=[
            pl.BlockSpec((1, c, flatpad), lambda bi, ii, ti: (bi, 0, 0)),
            pl.BlockSpec((1, c, flatpad), lambda bi, ii, ti: (bi, 0, 0)),
        ],
        out_specs=pl.BlockSpec((1, 1), lambda bi, ii, ti: (0, 0)),
        out_shape=jax.ShapeDtypeStruct((1, 1), jnp.float32),
        scratch_shapes=[
            pltpu.VMEM((d, npad), jnp.float32),
            pltpu.VMEM((d, npad), jnp.float32),
            pltpu.VMEM((npad, d + 2), jnp.bfloat16),
            pltpu.VMEM((d + 2, npad), jnp.bfloat16),
            pltpu.VMEM((BLK, 128), jnp.float32),
            pltpu.SMEM((1, 1), jnp.float32),
        ],
        compiler_params=pltpu.CompilerParams(
            dimension_semantics=("arbitrary", "arbitrary", "arbitrary"),
        ),
    )(xf, yf)
    return out[0, 0]
